# baseline (speedup 1.0000x reference)
import functools

import jax
import jax.numpy as jnp
from jax import lax
from jax.experimental import pallas as pl
from jax.experimental.pallas import tpu as pltpu
from jax.experimental.pallas import tpu_sc as plsc

_N = 5000
_FLAT = _N * 2
_NW = 32
_L = 16
_PAD = 10240
_CHUNK = _PAD // _NW
_VECS = _CHUNK // _L


def _sc_body(t_hbm, ac_hbm, om_hbm, tb_hbm, nz_hbm, out_hbm,
             t_v, c1_v, c2_v, tb_v, nz_v, o_v, sem):
    wid = lax.axis_index("s") * 2 + lax.axis_index("c")
    base = wid * _CHUNK
    pltpu.sync_copy(t_hbm, t_v)
    pltpu.sync_copy(tb_hbm.at[pl.ds(base, _CHUNK)], tb_v)
    pltpu.sync_copy(nz_hbm.at[pl.ds(base, _CHUNK)], nz_v)
    pltpu.async_copy(ac_hbm.at[t_v], c1_v, sem).wait()
    pltpu.async_copy(om_hbm.at[t_v], c2_v, sem).wait()
    c1 = c1_v[...]
    c2 = c2_v[...]
    for j in range(_VECS):
        sl = pl.ds(j * _L, _L)
        x = tb_v[sl] * 2.0 - 1.0
        y = c1 * x + c2 * nz_v[sl]
        y = jnp.clip(y, -1.0, 1.0)
        o_v[sl] = (y + 1.0) * 0.5
    pltpu.sync_copy(o_v, out_hbm.at[pl.ds(base, _CHUNK)])


@jax.jit
def kernel(audio, true_boxes, sqrt_alphas_cumprod, sqrt_one_minus_alphas_cumprod, noise, t):
    del audio
    tb = jnp.pad(true_boxes.reshape(_FLAT), (0, _PAD - _FLAT))
    nz = jnp.pad(noise.reshape(_FLAT), (0, _PAD - _FLAT))
    t16 = jnp.broadcast_to(t.astype(jnp.int32), (_L,))
    run = pl.kernel(
        _sc_body,
        out_type=jax.ShapeDtypeStruct((_PAD,), jnp.float32),
        mesh=plsc.VectorSubcoreMesh(core_axis_name="c", subcore_axis_name="s"),
        scratch_types=[
            pltpu.VMEM((_L,), jnp.int32),
            pltpu.VMEM((_L,), jnp.float32),
            pltpu.VMEM((_L,), jnp.float32),
            pltpu.VMEM((_CHUNK,), jnp.float32),
            pltpu.VMEM((_CHUNK,), jnp.float32),
            pltpu.VMEM((_CHUNK,), jnp.float32),
            pltpu.SemaphoreType.DMA,
        ],
    )
    out = run(t16, sqrt_alphas_cumprod, sqrt_one_minus_alphas_cumprod, tb, nz)
    return out[:_FLAT].reshape(_N, 2)


# speedup vs baseline: 1.0495x; 1.0495x over previous
import jax
import jax.numpy as jnp
from jax import lax
from jax.experimental import pallas as pl
from jax.experimental.pallas import tpu as pltpu
from jax.experimental.pallas import tpu_sc as plsc

_N = 5000
_FLAT = _N * 2
_L = 16
_ACTIVE = 25
_CHUNK = _FLAT // _ACTIVE
_VECS = _CHUNK // _L


def _sc_body(t_hbm, ac_hbm, om_hbm, tb_hbm, nz_hbm, out_hbm,
             t_v, c1_v, c2_v, tb_v, nz_v, o_v,
             sem_a, sem_b, sem_c, sem_d):
    wid = lax.axis_index("s") * 2 + lax.axis_index("c")

    @pl.when(wid < _ACTIVE)
    def _():
        base = wid * _CHUNK
        cp_tb = pltpu.async_copy(tb_hbm.at[pl.ds(base, _CHUNK)], tb_v, sem_a)
        cp_nz = pltpu.async_copy(nz_hbm.at[pl.ds(base, _CHUNK)], nz_v, sem_b)
        cp_t = pltpu.async_copy(t_hbm, t_v, sem_c)
        cp_t.wait()
        tsplat = t_v[...]
        cp_c1 = pltpu.async_copy(ac_hbm.at[tsplat], c1_v, sem_d)
        cp_c2 = pltpu.async_copy(om_hbm.at[tsplat], c2_v, sem_d)
        cp_c1.wait()
        cp_c2.wait()
        cp_tb.wait()
        cp_nz.wait()
        c1 = c1_v[...]
        c2 = c2_v[...]
        for j in range(_VECS):
            sl = pl.ds(j * _L, _L)
            x = tb_v[sl] * 2.0 - 1.0
            y = c1 * x + c2 * nz_v[sl]
            y = jnp.clip(y, -1.0, 1.0)
            o_v[sl] = (y + 1.0) * 0.5
        pltpu.sync_copy(o_v, out_hbm.at[pl.ds(base, _CHUNK)])


@jax.jit
def kernel(audio, true_boxes, sqrt_alphas_cumprod, sqrt_one_minus_alphas_cumprod, noise, t):
    del audio
    tb = true_boxes.reshape(_FLAT)
    nz = noise.reshape(_FLAT)
    run = pl.kernel(
        _sc_body,
        out_type=jax.ShapeDtypeStruct((_FLAT,), jnp.float32),
        mesh=plsc.VectorSubcoreMesh(core_axis_name="c", subcore_axis_name="s"),
        scratch_types=[
            pltpu.VMEM((_L,), jnp.int32),
            pltpu.VMEM((_L,), jnp.float32),
            pltpu.VMEM((_L,), jnp.float32),
            pltpu.VMEM((_CHUNK,), jnp.float32),
            pltpu.VMEM((_CHUNK,), jnp.float32),
            pltpu.VMEM((_CHUNK,), jnp.float32),
            pltpu.SemaphoreType.DMA,
            pltpu.SemaphoreType.DMA,
            pltpu.SemaphoreType.DMA,
            pltpu.SemaphoreType.DMA,
        ],
    )
    t16 = jnp.broadcast_to(t.astype(jnp.int32), (_L,))
    out = run(t16, sqrt_alphas_cumprod,
              sqrt_one_minus_alphas_cumprod, tb, nz)
    return out.reshape(_N, 2)


# speedup vs baseline: 1.1142x; 1.0617x over previous
import jax
import jax.numpy as jnp
from jax import lax
from jax.experimental import pallas as pl
from jax.experimental.pallas import tpu as pltpu
from jax.experimental.pallas import tpu_sc as plsc

_N = 5000
_FLAT = _N * 2
_L = 16
_NW = 16
_PAD = 10240
_CHUNK = _PAD // _NW
_VECS = _CHUNK // _L


def _sc_body(t_hbm, ac_hbm, om_hbm, tb_hbm, nz_hbm, out_hbm,
             t_v, c1_v, c2_v, tb_v, nz_v, o_v,
             sem_a, sem_b, sem_c, sem_d):
    wid = lax.axis_index("s")
    base = wid * _CHUNK
    cp_tb = pltpu.async_copy(tb_hbm.at[pl.ds(base, _CHUNK)], tb_v, sem_a)
    cp_nz = pltpu.async_copy(nz_hbm.at[pl.ds(base, _CHUNK)], nz_v, sem_b)
    cp_t = pltpu.async_copy(t_hbm, t_v, sem_c)
    cp_t.wait()
    tsplat = t_v[...]
    cp_c1 = pltpu.async_copy(ac_hbm.at[tsplat], c1_v, sem_d)
    cp_c2 = pltpu.async_copy(om_hbm.at[tsplat], c2_v, sem_d)
    cp_c1.wait()
    cp_c2.wait()
    cp_tb.wait()
    cp_nz.wait()
    c1 = c1_v[...]
    c2 = c2_v[...]
    for j in range(_VECS):
        sl = pl.ds(j * _L, _L)
        x = tb_v[sl] * 2.0 - 1.0
        y = c1 * x + c2 * nz_v[sl]
        y = jnp.clip(y, -1.0, 1.0)
        o_v[sl] = (y + 1.0) * 0.5
    pltpu.sync_copy(o_v, out_hbm.at[pl.ds(base, _CHUNK)])


@jax.jit
def kernel(audio, true_boxes, sqrt_alphas_cumprod, sqrt_one_minus_alphas_cumprod, noise, t):
    del audio
    tb = jnp.pad(true_boxes.reshape(_FLAT), (0, _PAD - _FLAT))
    nz = jnp.pad(noise.reshape(_FLAT), (0, _PAD - _FLAT))
    t16 = jnp.broadcast_to(t.astype(jnp.int32), (_L,))
    run = pl.kernel(
        _sc_body,
        out_type=jax.ShapeDtypeStruct((_PAD,), jnp.float32),
        mesh=plsc.VectorSubcoreMesh(core_axis_name="c", subcore_axis_name="s",
                                    num_cores=1),
        scratch_types=[
            pltpu.VMEM((_L,), jnp.int32),
            pltpu.VMEM((_L,), jnp.float32),
            pltpu.VMEM((_L,), jnp.float32),
            pltpu.VMEM((_CHUNK,), jnp.float32),
            pltpu.VMEM((_CHUNK,), jnp.float32),
            pltpu.VMEM((_CHUNK,), jnp.float32),
            pltpu.SemaphoreType.DMA,
            pltpu.SemaphoreType.DMA,
            pltpu.SemaphoreType.DMA,
            pltpu.SemaphoreType.DMA,
        ],
    )
    out = run(t16, sqrt_alphas_cumprod, sqrt_one_minus_alphas_cumprod, tb, nz)
    return out[:_FLAT].reshape(_N, 2)


# speedup vs baseline: 1.1163x; 1.0019x over previous
import jax
import jax.numpy as jnp
from jax import lax
from jax.experimental import pallas as pl
from jax.experimental.pallas import tpu as pltpu
from jax.experimental.pallas import tpu_sc as plsc

_N = 5000
_FLAT = _N * 2
_L = 16
_NW = 16
_PAD = 10240
_CHUNK = _PAD // _NW
_VECS = _CHUNK // _L
_HALF = _CHUNK // 2


def _sc_body(t_hbm, ac_hbm, om_hbm, tb_hbm, nz_hbm, out_hbm,
             t_v, c1_v, c2_v, tb_v, nz_v, o_v,
             sem_in, sem_g, sem_out):
    wid = lax.axis_index("s")
    base = wid * _CHUNK
    cp_t = pltpu.async_copy(t_hbm, t_v, sem_g)
    cp_tb = pltpu.async_copy(tb_hbm.at[pl.ds(base, _CHUNK)], tb_v, sem_in)
    cp_nz = pltpu.async_copy(nz_hbm.at[pl.ds(base, _CHUNK)], nz_v, sem_in)
    cp_t.wait()
    tsplat = t_v[...]
    cp_c1 = pltpu.async_copy(ac_hbm.at[tsplat], c1_v, sem_g)
    cp_c2 = pltpu.async_copy(om_hbm.at[tsplat], c2_v, sem_g)
    cp_c1.wait()
    cp_c2.wait()
    cp_tb.wait()
    cp_nz.wait()
    c1 = c1_v[...]
    c2 = c2_v[...]
    cp_lo = None
    for j in range(_VECS):
        sl = pl.ds(j * _L, _L)
        x = tb_v[sl] * 2.0 - 1.0
        y = c1 * x + c2 * nz_v[sl]
        y = jnp.clip(y, -1.0, 1.0)
        o_v[sl] = (y + 1.0) * 0.5
        if j == _VECS // 2 - 1:
            cp_lo = pltpu.async_copy(
                o_v.at[pl.ds(0, _HALF)],
                out_hbm.at[pl.ds(base, _HALF)], sem_out)
    cp_hi = pltpu.async_copy(
        o_v.at[pl.ds(_HALF, _HALF)],
        out_hbm.at[pl.ds(base + _HALF, _HALF)], sem_out)
    cp_lo.wait()
    cp_hi.wait()


@jax.jit
def kernel(audio, true_boxes, sqrt_alphas_cumprod, sqrt_one_minus_alphas_cumprod, noise, t):
    del audio
    tb = jnp.pad(true_boxes.reshape(_FLAT), (0, _PAD - _FLAT))
    nz = jnp.pad(noise.reshape(_FLAT), (0, _PAD - _FLAT))
    t16 = jnp.broadcast_to(t.astype(jnp.int32), (_L,))
    run = pl.kernel(
        _sc_body,
        out_type=jax.ShapeDtypeStruct((_PAD,), jnp.float32),
        mesh=plsc.VectorSubcoreMesh(core_axis_name="c", subcore_axis_name="s",
                                    num_cores=1),
        scratch_types=[
            pltpu.VMEM((_L,), jnp.int32),
            pltpu.VMEM((_L,), jnp.float32),
            pltpu.VMEM((_L,), jnp.float32),
            pltpu.VMEM((_CHUNK,), jnp.float32),
            pltpu.VMEM((_CHUNK,), jnp.float32),
            pltpu.VMEM((_CHUNK,), jnp.float32),
            pltpu.SemaphoreType.DMA,
            pltpu.SemaphoreType.DMA,
            pltpu.SemaphoreType.DMA,
        ],
    )
    out = run(t16, sqrt_alphas_cumprod, sqrt_one_minus_alphas_cumprod, tb, nz)
    return out[:_FLAT].reshape(_N, 2)
